# baseline (speedup 1.0000x reference)
import functools

import jax
import jax.numpy as jnp
from jax import lax
from jax.experimental import pallas as pl
from jax.experimental.pallas import tpu as pltpu
from jax.experimental.pallas import tpu_sc as plsc

BATCH = 16384
DIM = 64
NC = 2
NS = 16
NW = NC * NS
BPW = BATCH // NW
L = 16

NIDS = 2 * BATCH
TRASH = NIDS
SENT = 0x7FFFFFFF

NTILES = 1000000 // 8
CT = 16
CROWS = CT * 8
NCH = -(-NTILES // CT)
NCH_BASE = NCH // NW
NCH_REM = NCH % NW
WCAP = NIDS + L

RCHUNK = 128
NRCH = BPW // RCHUNK

_mesh = plsc.VectorSubcoreMesh(core_axis_name="c", subcore_axis_name="s")


@functools.partial(
    pl.kernel,
    mesh=_mesh,
    out_type=(
        jax.ShapeDtypeStruct((NIDS + 8, 2 * DIM), jnp.float32),
        jax.ShapeDtypeStruct((BATCH, 2 * DIM), jnp.float32),
    ),
    compiler_params=pltpu.CompilerParams(needs_layout_passes=False),
    scratch_types=[
        pltpu.VMEM((WCAP,), jnp.int32),
        pltpu.VMEM((NRCH, RCHUNK), jnp.int32),
        pltpu.VMEM((RCHUNK, 2 * DIM), jnp.float32),
        pltpu.VMEM((CT, 8, DIM), jnp.float32),
        pltpu.VMEM((CT, 8, DIM), jnp.float32),
        pltpu.VMEM((CT, 8, DIM), jnp.float32),
        pltpu.VMEM((CT, 8, DIM), jnp.float32),
        pltpu.VMEM((L, 2 * DIM), jnp.float32),
        pltpu.SemaphoreType.DMA,
        pltpu.SemaphoreType.DMA,
        pltpu.SemaphoreType.DMA,
    ],
)
def _sc_gather(rs_hbm, wl_hbm, ent_re_hbm, ent_im_hbm, rel_hbm,
               rows_hbm, relrows_hbm,
               wl, r2, rbuf, reA, imA, reB, imB, stage, sem, semA, semB):
    wid = lax.axis_index("s") * NC + lax.axis_index("c")
    nch = NCH_BASE + (wid < NCH_REM).astype(jnp.int32)
    g0 = wid * NCH_BASE + jnp.minimum(wid, NCH_REM)
    base_row = g0 * CROWS

    pltpu.sync_copy(wl_hbm.at[wid], wl)

    rbase = wid * BPW
    for c in range(NRCH):
        pltpu.sync_copy(rs_hbm.at[pl.ds(rbase + c * RCHUNK, RCHUNK)], r2.at[c])

    def rel_chunk(c, _):
        cp = pltpu.async_copy(rel_hbm.at[r2.at[c]], rbuf, sem)
        cp.wait()
        pltpu.sync_copy(
            rbuf, relrows_hbm.at[pl.ds(rbase + c * RCHUNK, RCHUNK)])
        return 0

    lax.fori_loop(0, NRCH, rel_chunk, 0)

    lane = lax.iota(jnp.int32, L)

    def issue(k, re_buf, im_buf, dsem):
        g = g0 + jnp.minimum(k, nch - 1)
        t0 = jnp.minimum(g * CT, NTILES - CT)
        a = pltpu.async_copy(ent_re_hbm.at[pl.ds(t0, CT)], re_buf, dsem)
        b = pltpu.async_copy(ent_im_hbm.at[pl.ds(t0, CT)], im_buf, dsem)
        return a, b

    def process(k, re_buf, im_buf, ptr):
        g = g0 + jnp.minimum(k, nch - 1)
        t0 = jnp.minimum(g * CT, NTILES - CT)
        lo = t0 * 8 - base_row
        hi = lo + CROWS

        def cond(s):
            return s[1]

        def body(s):
            p, _ = s
            packed = wl[pl.ds(p, L)]
            h = packed >> 16
            pos = packed & 0xFFFF
            mask = (h >= lo) & (h < hi)

            @pl.when(jnp.any(mask))
            def _():
                r = jnp.clip(h - lo, 0, CROWS - 1)
                rt = r >> 3
                rsub = r & 7
                spos = jnp.where(mask, pos, TRASH)
                for d in range(DIM):
                    dv = lane * 0 + d
                    a = plsc.load_gather(re_buf, [rt, rsub, dv])
                    b = plsc.load_gather(im_buf, [rt, rsub, dv])
                    plsc.store_scatter(stage, [lane, dv], a)
                    plsc.store_scatter(stage, [lane, dv + DIM], b)
                pltpu.async_copy(stage, rows_hbm.at[spos], sem).wait()

            below = jnp.logical_not(jnp.any(h >= hi))
            return p + jnp.where(below, L, 0), below

        ptr, _ = lax.while_loop(cond, body, (ptr, True))
        return ptr

    npad = nch + (nch & 1)
    issue(0, reA, imA, semA)

    def pair(i, ptr):
        k0 = 2 * i
        a1, b1 = issue(k0 + 1, reB, imB, semB)
        pltpu.make_async_copy(ent_re_hbm.at[pl.ds(0, CT)], reA, semA).wait()
        pltpu.make_async_copy(ent_im_hbm.at[pl.ds(0, CT)], imA, semA).wait()
        ptr = process(k0, reA, imA, ptr)

        @pl.when(k0 + 2 < npad)
        def _():
            issue(k0 + 2, reA, imA, semA)

        a1.wait()
        b1.wait()
        ptr = process(k0 + 1, reB, imB, ptr)
        return ptr

    lax.fori_loop(0, npad // 2, pair, 0)


_TB = 2048


def _score_body(h_ref, t_ref, rel_ref, out_ref):
    re_h = h_ref[:, :DIM]
    im_h = h_ref[:, DIM:]
    re_t = t_ref[:, :DIM]
    im_t = t_ref[:, DIM:]
    r_re = rel_ref[:, :DIM]
    r_im = rel_ref[:, DIM:]
    f = r_re * (re_h * re_t + im_h * im_t) + r_im * (re_h * im_t - im_h * re_t)
    out_ref[...] = jnp.sum(f, axis=1, keepdims=True)


def _tc_score(rows, relrows):
    nb = BATCH // _TB
    return pl.pallas_call(
        _score_body,
        grid=(nb,),
        in_specs=[
            pl.BlockSpec((_TB, 2 * DIM), lambda i: (i, 0)),
            pl.BlockSpec((_TB, 2 * DIM), lambda i, nb=nb: (nb + i, 0)),
            pl.BlockSpec((_TB, 2 * DIM), lambda i: (i, 0)),
        ],
        out_specs=pl.BlockSpec((_TB, 1), lambda i: (i, 0)),
        out_shape=jax.ShapeDtypeStruct((BATCH, 1), jnp.float32),
    )(rows, rows, relrows)


def _build_worklist(hs, ts):
    ids = jnp.concatenate([hs, ts])
    order = jnp.argsort(ids).astype(jnp.int32)
    srt = ids[order]
    w = jnp.arange(NW, dtype=jnp.int32)
    g0 = w * NCH_BASE + jnp.minimum(w, NCH_REM)
    ncw = NCH_BASE + (w < NCH_REM).astype(jnp.int32)
    starts = g0 * CROWS
    ends = jnp.minimum((g0 + ncw) * CROWS, 1000000)
    s = jnp.searchsorted(srt, starts, side="left").astype(jnp.int32)
    e = jnp.searchsorted(srt, ends, side="left").astype(jnp.int32)
    j = jnp.arange(WCAP, dtype=jnp.int32)
    idx2 = s[:, None] + j[None, :]
    valid = idx2 < e[:, None]
    safe = jnp.clip(idx2, 0, NIDS - 1)
    packed = ((srt[safe] - starts[:, None]) << 16) | order[safe]
    return jnp.where(valid, packed, SENT)


def kernel(hs, rs, ts, ent_re, ent_im, rel_re, rel_im):
    ent_re3 = ent_re.reshape(NTILES, 8, DIM)
    ent_im3 = ent_im.reshape(NTILES, 8, DIM)
    rel_cat = jnp.concatenate([rel_re, rel_im], axis=1)
    wl = _build_worklist(hs, ts)
    rows, relrows = _sc_gather(rs, wl, ent_re3, ent_im3, rel_cat)
    out = _tc_score(rows, relrows)
    return out.reshape(BATCH)


# speedup vs baseline: 29.6955x; 29.6955x over previous
import functools

import jax
import jax.numpy as jnp
import numpy as np
from jax import lax
from jax.experimental import pallas as pl
from jax.experimental.pallas import tpu as pltpu
from jax.experimental.pallas import tpu_sc as plsc

BATCH = 16384
DIM = 64
NC = 2
NS = 16
NW = NC * NS
BPW = BATCH // NW
CHUNK = 128
NCHUNK = BPW // CHUNK
L = 16
G = CHUNK // L

_mesh = plsc.VectorSubcoreMesh(core_axis_name="c", subcore_axis_name="s")

_GATHER_DNUMS = lax.GatherDimensionNumbers(
    offset_dims=(), collapsed_slice_dims=(0,), start_index_map=(0,))


def _permute(x, idx):
    return lax.gather(x, idx[:, None], _GATHER_DNUMS, slice_sizes=(1,),
                      mode=lax.GatherScatterMode.PROMISE_IN_BOUNDS)


def _allsum(x, lane):
    for m in (8, 4, 2, 1):
        x = x + _permute(x, lane ^ m)
    return x


@functools.partial(
    pl.kernel,
    mesh=_mesh,
    out_type=jax.ShapeDtypeStruct((BATCH,), jnp.float32),
    compiler_params=pltpu.CompilerParams(needs_layout_passes=False),
    scratch_types=[
        pltpu.VMEM((NCHUNK, CHUNK), jnp.int32),
        pltpu.VMEM((NCHUNK, CHUNK), jnp.int32),
        pltpu.VMEM((NCHUNK, CHUNK), jnp.int32),
        pltpu.VMEM((CHUNK, 2 * DIM), jnp.float32),
        pltpu.VMEM((CHUNK, 2 * DIM), jnp.float32),
        pltpu.VMEM((CHUNK, 2 * DIM), jnp.float32),
        pltpu.VMEM((BPW,), jnp.float32),
        pltpu.SemaphoreType.DMA,
    ],
)
def _complex_sc(hs_hbm, rs_hbm, ts_hbm, ent_hbm, rel_hbm, out_hbm,
                h2, t2, r2, ch, ct, cr, out_v, sem):
    wid = lax.axis_index("s") * NC + lax.axis_index("c")
    base = wid * BPW
    for c in range(NCHUNK):
        off = base + c * CHUNK
        pltpu.sync_copy(hs_hbm.at[pl.ds(off, CHUNK)], h2.at[c])
        pltpu.sync_copy(ts_hbm.at[pl.ds(off, CHUNK)], t2.at[c])
        pltpu.sync_copy(rs_hbm.at[pl.ds(off, CHUNK)], r2.at[c])

    def chunk(c, _):
        copies = [
            pltpu.async_copy(ent_hbm.at[h2.at[c]], ch, sem),
            pltpu.async_copy(ent_hbm.at[t2.at[c]], ct, sem),
            pltpu.async_copy(rel_hbm.at[r2.at[c]], cr, sem),
        ]
        for cp in copies:
            cp.wait()

        def group(g, _):
            lane = lax.iota(jnp.int32, L)
            scores = jnp.zeros((L,), jnp.float32)
            for k in range(L):
                i = g * L + k
                acc = jnp.zeros((L,), jnp.float32)
                for j in range(DIM // L):
                    re_sl = pl.ds(j * L, L)
                    im_sl = pl.ds(DIM + j * L, L)
                    a = ch[i, re_sl]
                    b = ch[i, im_sl]
                    u = ct[i, re_sl]
                    v = ct[i, im_sl]
                    p = cr[i, re_sl]
                    q = cr[i, im_sl]
                    acc = acc + p * (a * u + b * v) + q * (a * v - b * u)
                scores = jnp.where(lane == k, _allsum(acc, lane), scores)
            out_v[pl.ds(c * CHUNK + g * L, L)] = scores
            return 0

        lax.fori_loop(0, G, group, 0)
        return 0

    lax.fori_loop(0, NCHUNK, chunk, 0)
    pltpu.sync_copy(out_v, out_hbm.at[pl.ds(base, BPW)])


_P_LEFT = np.concatenate(
    [np.eye(DIM, dtype=np.float32), np.zeros((DIM, DIM), np.float32)], axis=1)
_P_RIGHT = np.concatenate(
    [np.zeros((DIM, DIM), np.float32), np.eye(DIM, dtype=np.float32)], axis=1)


def kernel(hs, rs, ts, ent_re, ent_im, rel_re, rel_im):
    ent_cat = (ent_re @ jnp.asarray(_P_LEFT)
               + ent_im @ jnp.asarray(_P_RIGHT))
    rel_cat = jnp.concatenate([rel_re, rel_im], axis=1)
    return _complex_sc(hs, rs, ts, ent_cat, rel_cat)


# speedup vs baseline: 38.4189x; 1.2938x over previous
import functools

import jax
import jax.numpy as jnp
import numpy as np
from jax import lax
from jax.experimental import pallas as pl
from jax.experimental.pallas import tpu as pltpu
from jax.experimental.pallas import tpu_sc as plsc

BATCH = 16384
DIM = 64
NC = 2
NS = 16
NW = NC * NS
BPW = BATCH // NW
CHUNK = 128
NCHUNK = BPW // CHUNK
L = 16
G = CHUNK // L

_mesh = plsc.VectorSubcoreMesh(core_axis_name="c", subcore_axis_name="s")

_GATHER_DNUMS = lax.GatherDimensionNumbers(
    offset_dims=(), collapsed_slice_dims=(0,), start_index_map=(0,))


def _permute(x, idx):
    return lax.gather(x, idx[:, None], _GATHER_DNUMS, slice_sizes=(1,),
                      mode=lax.GatherScatterMode.PROMISE_IN_BOUNDS)


def _allsum(x, lane):
    for m in (8, 4, 2, 1):
        x = x + _permute(x, lane ^ m)
    return x


@functools.partial(
    pl.kernel,
    mesh=_mesh,
    out_type=jax.ShapeDtypeStruct((BATCH,), jnp.float32),
    compiler_params=pltpu.CompilerParams(needs_layout_passes=False),
    scratch_types=[
        pltpu.VMEM((NCHUNK, CHUNK), jnp.int32),
        pltpu.VMEM((NCHUNK, CHUNK), jnp.int32),
        pltpu.VMEM((NCHUNK, CHUNK), jnp.int32),
        pltpu.VMEM((CHUNK, 2 * DIM), jnp.float32),
        pltpu.VMEM((CHUNK, 2 * DIM), jnp.float32),
        pltpu.VMEM((CHUNK, 2 * DIM), jnp.float32),
        pltpu.VMEM((CHUNK, 2 * DIM), jnp.float32),
        pltpu.VMEM((CHUNK, 2 * DIM), jnp.float32),
        pltpu.VMEM((BPW,), jnp.float32),
        pltpu.SemaphoreType.DMA,
    ],
)
def _complex_sc(hs_hbm, rs_hbm, ts_hbm, entre_hbm, entim_hbm, rel_hbm,
                out_hbm, h2, t2, r2, chre, chim, ctre, ctim, cr, out_v, sem):
    wid = lax.axis_index("s") * NC + lax.axis_index("c")
    base = wid * BPW
    for c in range(NCHUNK):
        off = base + c * CHUNK
        pltpu.sync_copy(hs_hbm.at[pl.ds(off, CHUNK)], h2.at[c])
        pltpu.sync_copy(ts_hbm.at[pl.ds(off, CHUNK)], t2.at[c])
        pltpu.sync_copy(rs_hbm.at[pl.ds(off, CHUNK)], r2.at[c])

    def chunk(c, _):
        copies = [
            pltpu.async_copy(entre_hbm.at[h2.at[c]], chre, sem),
            pltpu.async_copy(entim_hbm.at[h2.at[c]], chim, sem),
            pltpu.async_copy(entre_hbm.at[t2.at[c]], ctre, sem),
            pltpu.async_copy(entim_hbm.at[t2.at[c]], ctim, sem),
            pltpu.async_copy(rel_hbm.at[r2.at[c]], cr, sem),
        ]
        for cp in copies:
            cp.wait()

        def group(g, _):
            lane = lax.iota(jnp.int32, L)
            scores = jnp.zeros((L,), jnp.float32)
            for k in range(L):
                i = g * L + k
                acc = jnp.zeros((L,), jnp.float32)
                for j in range(DIM // L):
                    re_sl = pl.ds(j * L, L)
                    im_sl = pl.ds(DIM + j * L, L)
                    a = chre[i, re_sl]
                    b = chim[i, im_sl]
                    u = ctre[i, re_sl]
                    v = ctim[i, im_sl]
                    p = cr[i, re_sl]
                    q = cr[i, im_sl]
                    acc = acc + p * (a * u + b * v) + q * (a * v - b * u)
                scores = jnp.where(lane == k, _allsum(acc, lane), scores)
            out_v[pl.ds(c * CHUNK + g * L, L)] = scores
            return 0

        lax.fori_loop(0, G, group, 0)
        return 0

    lax.fori_loop(0, NCHUNK, chunk, 0)
    pltpu.sync_copy(out_v, out_hbm.at[pl.ds(base, BPW)])


_P_LEFT = np.concatenate(
    [np.eye(DIM, dtype=np.float32), np.zeros((DIM, DIM), np.float32)], axis=1)
_P_RIGHT = np.concatenate(
    [np.zeros((DIM, DIM), np.float32), np.eye(DIM, dtype=np.float32)], axis=1)


def kernel(hs, rs, ts, ent_re, ent_im, rel_re, rel_im):
    ent_re_w = ent_re @ jnp.asarray(_P_LEFT)
    ent_im_w = ent_im @ jnp.asarray(_P_RIGHT)
    rel_cat = jnp.concatenate([rel_re, rel_im], axis=1)
    return _complex_sc(hs, rs, ts, ent_re_w, ent_im_w, rel_cat)


# speedup vs baseline: 38.7302x; 1.0081x over previous
import functools

import jax
import jax.numpy as jnp
import numpy as np
from jax import lax
from jax.experimental import pallas as pl
from jax.experimental.pallas import tpu as pltpu
from jax.experimental.pallas import tpu_sc as plsc

BATCH = 16384
DIM = 64
NC = 2
NS = 16
NW = NC * NS
BPW = BATCH // NW
CHUNK = 64
NCHUNK = BPW // CHUNK
L = 16
G = CHUNK // L

_mesh = plsc.VectorSubcoreMesh(core_axis_name="c", subcore_axis_name="s")

_GATHER_DNUMS = lax.GatherDimensionNumbers(
    offset_dims=(), collapsed_slice_dims=(0,), start_index_map=(0,))


def _permute(x, idx):
    return lax.gather(x, idx[:, None], _GATHER_DNUMS, slice_sizes=(1,),
                      mode=lax.GatherScatterMode.PROMISE_IN_BOUNDS)


def _allsum(x, lane):
    for m in (8, 4, 2, 1):
        x = x + _permute(x, lane ^ m)
    return x


@functools.partial(
    pl.kernel,
    mesh=_mesh,
    out_type=jax.ShapeDtypeStruct((BATCH,), jnp.float32),
    compiler_params=pltpu.CompilerParams(needs_layout_passes=False),
    scratch_types=[
        pltpu.VMEM((NCHUNK, CHUNK), jnp.int32),
        pltpu.VMEM((NCHUNK, CHUNK), jnp.int32),
        pltpu.VMEM((NCHUNK, CHUNK), jnp.int32),
        pltpu.VMEM((CHUNK, 2 * DIM), jnp.float32),
        pltpu.VMEM((CHUNK, 2 * DIM), jnp.float32),
        pltpu.VMEM((CHUNK, 2 * DIM), jnp.float32),
        pltpu.VMEM((CHUNK, 2 * DIM), jnp.float32),
        pltpu.VMEM((CHUNK, 2 * DIM), jnp.float32),
        pltpu.VMEM((CHUNK, 2 * DIM), jnp.float32),
        pltpu.VMEM((CHUNK, 2 * DIM), jnp.float32),
        pltpu.VMEM((CHUNK, 2 * DIM), jnp.float32),
        pltpu.VMEM((CHUNK, 2 * DIM), jnp.float32),
        pltpu.VMEM((CHUNK, 2 * DIM), jnp.float32),
        pltpu.VMEM((BPW,), jnp.float32),
        pltpu.SemaphoreType.DMA,
        pltpu.SemaphoreType.DMA,
    ],
)
def _complex_sc(hs_hbm, rs_hbm, ts_hbm, entre_hbm, entim_hbm, rel_hbm,
                out_hbm, h2, t2, r2,
                a0, a1, a2, a3, a4, b0, b1, b2, b3, b4,
                out_v, semA, semB):
    wid = lax.axis_index("s") * NC + lax.axis_index("c")
    base = wid * BPW
    for c in range(NCHUNK):
        off = base + c * CHUNK
        pltpu.sync_copy(hs_hbm.at[pl.ds(off, CHUNK)], h2.at[c])
        pltpu.sync_copy(ts_hbm.at[pl.ds(off, CHUNK)], t2.at[c])
        pltpu.sync_copy(rs_hbm.at[pl.ds(off, CHUNK)], r2.at[c])

    setA = (a0, a1, a2, a3, a4)
    setB = (b0, b1, b2, b3, b4)

    def issue(c, bufs, dsem):
        pltpu.async_copy(entre_hbm.at[h2.at[c]], bufs[0], dsem)
        pltpu.async_copy(entim_hbm.at[h2.at[c]], bufs[1], dsem)
        pltpu.async_copy(entre_hbm.at[t2.at[c]], bufs[2], dsem)
        pltpu.async_copy(entim_hbm.at[t2.at[c]], bufs[3], dsem)
        pltpu.async_copy(rel_hbm.at[r2.at[c]], bufs[4], dsem)

    def drain(bufs, dsem):
        for dst in bufs:
            pltpu.make_async_copy(
                entre_hbm.at[pl.ds(0, CHUNK)], dst, dsem).wait()

    def process(c, bufs):
        chre, chim, ctre, ctim, cr = bufs

        def group(g, _):
            lane = lax.iota(jnp.int32, L)
            scores = jnp.zeros((L,), jnp.float32)
            for k in range(L):
                i = g * L + k
                acc = jnp.zeros((L,), jnp.float32)
                for j in range(DIM // L):
                    re_sl = pl.ds(j * L, L)
                    im_sl = pl.ds(DIM + j * L, L)
                    a = chre[i, re_sl]
                    b = chim[i, im_sl]
                    u = ctre[i, re_sl]
                    v = ctim[i, im_sl]
                    p = cr[i, re_sl]
                    q = cr[i, im_sl]
                    acc = acc + p * (a * u + b * v) + q * (a * v - b * u)
                scores = jnp.where(lane == k, _allsum(acc, lane), scores)
            out_v[pl.ds(c * CHUNK + g * L, L)] = scores
            return 0

        lax.fori_loop(0, G, group, 0)

    issue(0, setA, semA)

    def pair(i, _):
        c0 = 2 * i
        issue(c0 + 1, setB, semB)
        drain(setA, semA)
        process(c0, setA)

        @pl.when(c0 + 2 < NCHUNK)
        def _():
            issue(c0 + 2, setA, semA)

        drain(setB, semB)
        process(c0 + 1, setB)
        return 0

    lax.fori_loop(0, NCHUNK // 2, pair, 0)
    pltpu.sync_copy(out_v, out_hbm.at[pl.ds(base, BPW)])


_P_LEFT = np.concatenate(
    [np.eye(DIM, dtype=np.float32), np.zeros((DIM, DIM), np.float32)], axis=1)
_P_RIGHT = np.concatenate(
    [np.zeros((DIM, DIM), np.float32), np.eye(DIM, dtype=np.float32)], axis=1)


def kernel(hs, rs, ts, ent_re, ent_im, rel_re, rel_im):
    ent_re_w = ent_re @ jnp.asarray(_P_LEFT)
    ent_im_w = ent_im @ jnp.asarray(_P_RIGHT)
    rel_cat = jnp.concatenate([rel_re, rel_im], axis=1)
    return _complex_sc(hs, rs, ts, ent_re_w, ent_im_w, rel_cat)
